# baseline (speedup 1.0000x reference)
import jax
import jax.numpy as jnp
from jax.experimental import pallas as pl
from jax.experimental.pallas import tpu as pltpu

_BLOCK = 3136


def _body(ids_ref, x_ref, emb_ref, o_ref):
    row = emb_ref[ids_ref[0], :]
    o_ref[...] = x_ref[...] + row[None, :]


def kernel(lang, emb_weight, dataset_id):
    b, s, d = lang.shape
    n_vocab = emb_weight.shape[0]
    rows = b * s
    x = lang.reshape(rows, d)
    ids = jnp.asarray(dataset_id, jnp.int32).reshape(1)

    grid_spec = pltpu.PrefetchScalarGridSpec(
        num_scalar_prefetch=1,
        grid=(pl.cdiv(rows, _BLOCK),),
        in_specs=[
            pl.BlockSpec((_BLOCK, d), lambda i, ids: (i, 0)),
            pl.BlockSpec((n_vocab, d), lambda i, ids: (0, 0)),
        ],
        out_specs=pl.BlockSpec((_BLOCK, d), lambda i, ids: (i, 0)),
    )
    out = pl.pallas_call(
        _body,
        grid_spec=grid_spec,
        out_shape=jax.ShapeDtypeStruct((rows, d), lang.dtype),
        compiler_params=pltpu.CompilerParams(
            dimension_semantics=("parallel",),
        ),
    )(ids, x, emb_weight)
    return out.reshape(b, s, d)
